# baseline (speedup 1.0000x reference)
import functools

import jax
import jax.numpy as jnp
from jax import lax
from jax.experimental import pallas as pl
from jax.experimental.pallas import tpu as pltpu
from jax.experimental.pallas import tpu_sc as plsc

L = 16


@functools.lru_cache(maxsize=None)
def _build_sc_kernel(B, C, V, D):
    info = plsc.get_sparse_core_info()
    NC, NS = info.num_cores, info.num_subcores
    NW = NC * NS
    assert B % (NW * L) == 0
    BPW = B // NW
    CB = L
    NCH = BPW // CB
    NP = D // 2
    PB = 8
    NPB = NP // PB

    mesh = plsc.VectorSubcoreMesh(core_axis_name="c", subcore_axis_name="s")

    @functools.partial(
        pl.kernel,
        mesh=mesh,
        out_type=jax.ShapeDtypeStruct((B * C,), jnp.float32),
        compiler_params=pltpu.CompilerParams(
            needs_layout_passes=False,
            use_tc_tiling_on_sc=False,
        ),
        scratch_types=[
            pltpu.VMEM((CB,), jnp.int32),
            pltpu.VMEM((CB * C,), jnp.int32),
            pltpu.VMEM((CB, D // 2), jnp.int32),
            pltpu.VMEM((CB * C, D // 2), jnp.int32),
            pltpu.VMEM((CB * C,), jnp.float32),
            pltpu.SemaphoreType.DMA,
            pltpu.SemaphoreType.DMA,
        ],
    )
    def sc_kernel(center_hbm, ctx_hbm, wc_hbm, wk_hbm, out_hbm,
                  cidx, kidx, crows, krows, outv, sem1, sem2):
        wid = lax.axis_index("s") * NC + lax.axis_index("c")
        wbase = wid * BPW
        iota = lax.broadcasted_iota(jnp.int32, (L,), 0)
        zerov = jnp.zeros((L,), jnp.float32)

        def chunk_body(i, carry):
            base = wbase + i * CB
            pltpu.sync_copy(center_hbm.at[pl.ds(base, CB)], cidx)
            pltpu.sync_copy(ctx_hbm.at[pl.ds(base * C, CB * C)], kidx)
            h1 = pltpu.async_copy(wc_hbm.at[cidx], crows, sem1)
            h2 = pltpu.async_copy(wk_hbm.at[kidx], krows, sem2)
            h1.wait()
            h2.wait()

            def pblk_body(pblk, accs):
                p0 = pblk * PB
                cc = []
                for p in range(PB):
                    cpair = plsc.load_gather(crows, [iota, iota * 0 + (p0 + p)])
                    cc.append(plsc.unpack(
                        plsc.bitcast(cpair, jnp.bfloat16),
                        format=plsc.PackFormat.INTERLEAVED))
                new_accs = []
                for c in range(C):
                    a = accs[c]
                    rowc = iota * C + c
                    for p in range(PB):
                        kpair = plsc.load_gather(
                            krows, [rowc, iota * 0 + (p0 + p)])
                        ka, kb = plsc.unpack(
                            plsc.bitcast(kpair, jnp.bfloat16),
                            format=plsc.PackFormat.INTERLEAVED)
                        ca, cb = cc[p]
                        a = a + ca * ka + cb * kb
                    new_accs.append(a)
                return tuple(new_accs)

            accs = lax.fori_loop(0, NPB, pblk_body, (zerov,) * C)
            for c in range(C):
                plsc.store_scatter(outv, [iota * C + c], accs[c])
            pltpu.sync_copy(outv, out_hbm.at[pl.ds(base * C, CB * C)])
            return carry

        lax.fori_loop(0, NCH, chunk_body, 0)

    return sc_kernel


def kernel(center, context, W_center, W_context):
    B, C = context.shape
    V, D = W_center.shape
    center = jnp.asarray(center, jnp.int32)
    ctx_flat = jnp.asarray(context, jnp.int32).reshape(B * C)
    wc16 = lax.bitcast_convert_type(
        W_center.astype(jnp.bfloat16).reshape(V, D // 2, 2), jnp.int32)
    wk16 = lax.bitcast_convert_type(
        W_context.astype(jnp.bfloat16).reshape(V, D // 2, 2), jnp.int32)
    sc = _build_sc_kernel(B, C, V, D)
    out_flat = sc(center, ctx_flat, wc16, wk16)
    return out_flat.reshape(B, C)


# speedup vs baseline: 1.3796x; 1.3796x over previous
import functools

import jax
import jax.numpy as jnp
from jax import lax
from jax.experimental import pallas as pl
from jax.experimental.pallas import tpu as pltpu
from jax.experimental.pallas import tpu_sc as plsc

L = 16


@functools.lru_cache(maxsize=None)
def _build_sc_kernel(B, C, V, D):
    info = plsc.get_sparse_core_info()
    NC, NS = info.num_cores, info.num_subcores
    NW = NC * NS
    assert B % (NW * L) == 0
    BPW = B // NW
    CB = L
    NCH = BPW // CB
    NP = D // 2
    PB = 8
    NPB = NP // PB

    mesh = plsc.VectorSubcoreMesh(core_axis_name="c", subcore_axis_name="s")

    @functools.partial(
        pl.kernel,
        mesh=mesh,
        out_type=jax.ShapeDtypeStruct((B * C,), jnp.float32),
        compiler_params=pltpu.CompilerParams(
            needs_layout_passes=False,
            use_tc_tiling_on_sc=False,
        ),
        scratch_types=[
            pltpu.VMEM((CB,), jnp.int32),
            pltpu.VMEM((CB * C,), jnp.int32),
            pltpu.VMEM((CB, D), jnp.float32),
            pltpu.VMEM((CB * C, D // 2), jnp.int32),
            pltpu.VMEM((CB * C,), jnp.float32),
            pltpu.SemaphoreType.DMA,
            pltpu.SemaphoreType.DMA,
        ],
    )
    def sc_kernel(center_hbm, ctx_hbm, wc_hbm, wk_hbm, out_hbm,
                  cidx, kidx, crows, krows, outv, sem1, sem2):
        wid = lax.axis_index("s") * NC + lax.axis_index("c")
        wbase = wid * BPW
        iota = lax.broadcasted_iota(jnp.int32, (L,), 0)
        zerov = jnp.zeros((L,), jnp.float32)

        def chunk_body(i, carry):
            base = wbase + i * CB
            pltpu.sync_copy(center_hbm.at[pl.ds(base, CB)], cidx)
            pltpu.sync_copy(ctx_hbm.at[pl.ds(base * C, CB * C)], kidx)
            h1 = pltpu.async_copy(wc_hbm.at[cidx], crows, sem1)
            h2 = pltpu.async_copy(wk_hbm.at[kidx], krows, sem2)
            h1.wait()
            h2.wait()

            def pblk_body(pblk, accs):
                p0 = pblk * PB
                cc = [
                    plsc.load_gather(crows, [iota, iota * 0 + (2 * (p0 + p))])
                    for p in range(PB)
                ] + [
                    plsc.load_gather(crows, [iota, iota * 0 + (2 * (p0 + p) + 1)])
                    for p in range(PB)
                ]
                new_accs = []
                for c in range(C):
                    a = accs[c]
                    rowc = iota * C + c
                    for p in range(PB):
                        kpair = plsc.load_gather(
                            krows, [rowc, iota * 0 + (p0 + p)])
                        ka, kb = plsc.unpack(
                            plsc.bitcast(kpair, jnp.bfloat16),
                            format=plsc.PackFormat.INTERLEAVED)
                        a = a + cc[p] * ka + cc[PB + p] * kb
                    new_accs.append(a)
                return tuple(new_accs)

            accs = lax.fori_loop(0, NPB, pblk_body, (zerov,) * C)
            for c in range(C):
                plsc.store_scatter(outv, [iota * C + c], accs[c])
            pltpu.sync_copy(outv, out_hbm.at[pl.ds(base * C, CB * C)])
            return carry

        lax.fori_loop(0, NCH, chunk_body, 0)

    return sc_kernel


def kernel(center, context, W_center, W_context):
    B, C = context.shape
    V, D = W_center.shape
    center = jnp.asarray(center, jnp.int32)
    ctx_flat = jnp.asarray(context, jnp.int32).reshape(B * C)
    wk16 = lax.optimization_barrier(W_context.astype(jnp.bfloat16))
    wk_i = lax.bitcast_convert_type(wk16.reshape(V, D // 2, 2), jnp.int32)
    sc = _build_sc_kernel(B, C, V, D)
    out_flat = sc(center, ctx_flat, W_center, wk_i)
    return out_flat.reshape(B, C)


# speedup vs baseline: 2.2145x; 1.6052x over previous
import functools

import jax
import jax.numpy as jnp
from jax import lax
from jax.experimental import pallas as pl
from jax.experimental.pallas import tpu as pltpu
from jax.experimental.pallas import tpu_sc as plsc

L = 16


@functools.lru_cache(maxsize=None)
def _build_sc_kernel(B, C, V, D):
    info = plsc.get_sparse_core_info()
    NC, NS = info.num_cores, info.num_subcores
    NW = NC * NS
    assert B % (NW * L) == 0
    BPW = B // NW
    CB = 2 * L
    NCH = BPW // CB
    NG = CB // L
    DB = 8
    NDB = D // DB

    mesh = plsc.VectorSubcoreMesh(core_axis_name="c", subcore_axis_name="s")

    @functools.partial(
        pl.kernel,
        mesh=mesh,
        out_type=jax.ShapeDtypeStruct((B * C,), jnp.float32),
        compiler_params=pltpu.CompilerParams(
            needs_layout_passes=False,
            use_tc_tiling_on_sc=False,
        ),
        scratch_types=[
            pltpu.VMEM((BPW,), jnp.int32),
            pltpu.VMEM((BPW * C,), jnp.int32),
            pltpu.VMEM((BPW * C,), jnp.float32),
            pltpu.VMEM((CB, D), jnp.float32),
            pltpu.VMEM((CB * C, D), jnp.float32),
            pltpu.VMEM((CB, D), jnp.float32),
            pltpu.VMEM((CB * C, D), jnp.float32),
            pltpu.SemaphoreType.DMA,
            pltpu.SemaphoreType.DMA,
            pltpu.SemaphoreType.DMA,
            pltpu.SemaphoreType.DMA,
        ],
    )
    def sc_kernel(center_hbm, ctx_hbm, wc_hbm, wk_hbm, out_hbm,
                  cidx, kidx, outv, crows0, krows0, crows1, krows1,
                  semc0, semk0, semc1, semk1):
        crows = (crows0, crows1)
        krows = (krows0, krows1)
        semc = (semc0, semc1)
        semk = (semk0, semk1)
        wid = lax.axis_index("s") * NC + lax.axis_index("c")
        wbase = wid * BPW
        iota = lax.broadcasted_iota(jnp.int32, (L,), 0)
        zerov = jnp.zeros((L,), jnp.float32)

        pltpu.sync_copy(center_hbm.at[pl.ds(wbase, BPW)], cidx)
        pltpu.sync_copy(ctx_hbm.at[pl.ds(wbase * C, BPW * C)], kidx)

        def dma_pair(s, i):
            hc = pltpu.make_async_copy(
                wc_hbm.at[cidx.at[pl.ds(i * CB, CB)]], crows[s], semc[s])
            hk = pltpu.make_async_copy(
                wk_hbm.at[kidx.at[pl.ds(i * CB * C, CB * C)]],
                krows[s], semk[s])
            return hc, hk

        def issue(s, i):
            hc, hk = dma_pair(s, i)
            hc.start()
            hk.start()

        issue(0, jnp.int32(0))
        issue(1, jnp.int32(1))

        def chunk_pair_body(i2, carry):
            for s in range(2):
                i = i2 * 2 + s
                hc, hk = dma_pair(s, i)
                hc.wait()
                hk.wait()
                cr, kr = crows[s], krows[s]
                obase = i * (CB * C)
                for g in range(NG):
                    def dblk_body(dblk, accs):
                        d0 = dblk * DB
                        cc = [
                            plsc.load_gather(
                                cr, [g * L + iota, iota * 0 + (d0 + d)])
                            for d in range(DB)
                        ]
                        new_accs = []
                        for c in range(C):
                            a = accs[c]
                            rowc = iota * C + (g * L * C + c)
                            for d in range(DB):
                                kcol = plsc.load_gather(
                                    kr, [rowc, iota * 0 + (d0 + d)])
                                a = a + cc[d] * kcol
                            new_accs.append(a)
                        return tuple(new_accs)

                    accs = lax.fori_loop(0, NDB, dblk_body, (zerov,) * C)
                    for c in range(C):
                        plsc.store_scatter(
                            outv, [iota * C + (obase + g * L * C + c)],
                            accs[c])
                @pl.when(i + 2 < NCH)
                def _():
                    issue(s, i + 2)
            return carry

        lax.fori_loop(0, NCH // 2, chunk_pair_body, 0)
        pltpu.sync_copy(outv, out_hbm.at[pl.ds(wbase * C, BPW * C)])

    return sc_kernel


def kernel(center, context, W_center, W_context):
    B, C = context.shape
    V, D = W_center.shape
    center = jnp.asarray(center, jnp.int32)
    ctx_flat = jnp.asarray(context, jnp.int32).reshape(B * C)
    sc = _build_sc_kernel(B, C, V, D)
    out_flat = sc(center, ctx_flat, W_center, W_context)
    return out_flat.reshape(B, C)
